# baseline (speedup 1.0000x reference)
import functools

import jax
import jax.numpy as jnp
from jax import lax
from jax.experimental import pallas as pl
from jax.experimental.pallas import tpu as pltpu
from jax.experimental.pallas import tpu_sc as plsc

_N = 10000
_H = 128
_NG = 64
_K = 3
_E = 320000
_L = 3
_EPS = 1e-5
_NP = 10240
_R = 400
_C = 512
_NRB = _N // _R
_NCT = _NP // _C
_NW = 32
_NSUB = 16
_CH = 128
_NEG = float('-inf')

_HI = jax.lax.Precision.HIGHEST


def _edges_per_worker(e):
    per = -(-e // _NW)
    nch = -(-per // _CH)
    nch += nch % 2
    return nch, _NW * nch * _CH


_NCH_E, _EPAD = _edges_per_worker(_E)
_NCH_EH, _EHPAD = _edges_per_worker(_E // 2)
_NCH_F, _FPAD = _edges_per_worker(_N * _K)


def _prep_body(x_ref, we_ref, be_ref, brow_ref, bcol_ref,
               h_ref, hn_ref, pm_ref, mt_ref):
    x = x_ref[...]
    h = jnp.dot(x, we_ref[...], preferred_element_type=jnp.float32,
                precision=jax.lax.Precision.DEFAULT) + be_ref[...]
    h_ref[...] = h
    nrm = jnp.sqrt(jnp.sum(h * h, axis=1, keepdims=True))
    hn_ref[...] = h / jnp.maximum(nrm, 1e-12)
    gids = lax.broadcasted_iota(jnp.int32, (_NG, _N), 0)
    m = (gids == bcol_ref[...]).astype(jnp.float32)
    cnt = jnp.sum(m, axis=1, keepdims=True)
    pm_ref[...] = m / jnp.maximum(cnt, 1.0)
    gidsr = lax.broadcasted_iota(jnp.int32, (_N, _NG), 1)
    mt_ref[...] = (gidsr == brow_ref[...]).astype(jnp.float32)


def _knn_body(s_ref, hn_ref, hnt_ref, brow_ref, bcol_ref, out_ref):
    rb = pl.program_id(0)
    lo = s_ref[0, rb]
    hi = s_ref[1, rb]
    c0 = lo // _C
    c1 = (hi + _C - 1) // _C
    rows = hn_ref[...]
    rbatch = brow_ref[...]
    bv = jnp.full((_R, _K), _NEG, jnp.float32)
    bi = lax.broadcasted_iota(jnp.int32, (_R, _K), 1)

    def tilestep(c, carry):
        bv, bi = carry
        base = c * _C
        cols = hnt_ref[c]
        sim = jnp.dot(rows, cols, preferred_element_type=jnp.float32,
                      precision=jax.lax.Precision.DEFAULT)
        cbatch = bcol_ref[c]
        sim = jnp.where(rbatch == cbatch, sim, _NEG)
        iot = lax.broadcasted_iota(jnp.int32, (_R, _C), 1) + base
        cv, ci = [], []
        for _ in range(_K):
            v = jnp.max(sim, axis=1, keepdims=True)
            sel = jnp.min(jnp.where(sim == v, iot, _NP), axis=1,
                          keepdims=True)
            cv.append(v)
            ci.append(sel)
            sim = jnp.where(iot == sel, _NEG, sim)
        allv = jnp.concatenate([bv] + cv, axis=1)
        alli = jnp.concatenate([bi] + ci, axis=1)
        nv, ni = [], []
        for _ in range(_K):
            v = jnp.max(allv, axis=1, keepdims=True)
            sel = jnp.min(jnp.where(allv == v, alli, _NP), axis=1,
                          keepdims=True)
            nv.append(v)
            ni.append(sel)
            hitm = alli == sel
            allv = jnp.where(hitm, _NEG, allv)
            alli = jnp.where(hitm, _NP, alli)
        return jnp.concatenate(nv, axis=1), jnp.concatenate(ni, axis=1)

    bv, bi = lax.fori_loop(c0, c1, tilestep, (bv, bi))
    out_ref[...] = jnp.concatenate(
        [bi, jnp.zeros((_R, _H - _K), jnp.int32)], axis=1)


def _dinv_from_deg(degp_ref):
    d = 1.0 + degp_ref[0][:_N] + degp_ref[1][:_N]
    return lax.rsqrt(d)


def _prescale_body(h_ref, w_ref, degp_ref, u_ref):
    hw = jnp.dot(h_ref[...], w_ref[...], preferred_element_type=jnp.float32,
                 precision=_HI)
    u = _dinv_from_deg(degp_ref) * hw
    u_ref[...] = jnp.concatenate(
        [u, jnp.zeros((_NP - _N, _H), jnp.float32)], axis=0)


def _graph_norm(h, pm_ref, mt_ref, w, b, ms):
    mean = jnp.dot(pm_ref[...], h, preferred_element_type=jnp.float32,
                   precision=_HI)
    out = h - jnp.dot(mt_ref[...], mean, preferred_element_type=jnp.float32,
                      precision=_HI) * ms
    var = jnp.dot(pm_ref[...], out * out, preferred_element_type=jnp.float32,
                  precision=_HI)
    inv = lax.rsqrt(var + _EPS)
    return w * out * jnp.dot(mt_ref[...], inv,
                             preferred_element_type=jnp.float32,
                             precision=_HI) + b


def _leaky(x):
    return jnp.where(x >= 0, x, 0.01 * x)


def _convsum_body(acca_ref, accb_ref, u_ref, degp_ref, bc_ref, conv_ref):
    s = (acca_ref[0][: _N] + acca_ref[1][: _N]
         + accb_ref[0][: _N] + accb_ref[1][: _N] + u_ref[: _N])
    conv_ref[...] = _dinv_from_deg(degp_ref) * s + bc_ref[...]


def _convf_body(accf_ref, hwf_ref, bf_ref, convf_ref):
    convf_ref[...] = 0.25 * (accf_ref[0][: _N] + accf_ref[1][: _N]
                             + hwf_ref[: _N]) + bf_ref[...]


def _gnorm_body(conv_ref, pm_ref, mt_ref, w_ref, b_ref, ms_ref, out_ref):
    out_ref[...] = _leaky(_graph_norm(conv_ref[...], pm_ref, mt_ref,
                                      w_ref[...], b_ref[...], ms_ref[...]))


def _matpad_body(h1_ref, wf_ref, hwf_ref):
    hwf = jnp.dot(h1_ref[...], wf_ref[...],
                  preferred_element_type=jnp.float32, precision=_HI)
    hwf_ref[...] = jnp.concatenate(
        [hwf, jnp.zeros((_NP - _N, _H), jnp.float32)], axis=0)


def _gate_body(h1_ref, f_ref, prev_ref, pm_ref, wg1_ref, wg2_ref, bg_ref,
               pool_in_ref, h_ref, pool_ref):
    h1 = h1_ref[...]
    f = f_ref[...]
    z = (jnp.dot(h1, wg1_ref[...], preferred_element_type=jnp.float32,
                 precision=_HI)
         + jnp.dot(f, wg2_ref[...], preferred_element_type=jnp.float32,
                   precision=_HI) + bg_ref[...])
    gate = 1.0 / (1.0 + jnp.exp(-z))
    h = gate * h1 + (1.0 - gate) * f + prev_ref[...]
    h_ref[...] = h
    pool = jnp.dot(pm_ref[...], h, preferred_element_type=jnp.float32,
                   precision=_HI)
    pool_ref[...] = pool_in_ref[...] + pool


def _sc_mesh():
    return plsc.VectorSubcoreMesh(core_axis_name="c", subcore_axis_name="s",
                                  num_cores=2, num_subcores=_NSUB)


_ROWS_PER_SUB = _NP // _NSUB


def _sc_deg_kernel(dst_hbm, ones_hbm, zeros_hbm, out_hbm, idx_v, ones_v,
                   acc_sh):
    cid = lax.axis_index("c")
    sid = lax.axis_index("s")
    w = cid * _NSUB + sid
    pltpu.sync_copy(dst_hbm.at[w], idx_v)
    pltpu.sync_copy(ones_hbm, ones_v)
    pltpu.sync_copy(zeros_hbm.at[pl.ds(sid * _ROWS_PER_SUB, _ROWS_PER_SUB)],
                    acc_sh.at[pl.ds(sid * _ROWS_PER_SUB, _ROWS_PER_SUB)])
    plsc.subcore_barrier()

    def chunk(ch, carry):
        pltpu.sync_copy(ones_v, acc_sh.at[idx_v.at[ch]], add=True)
        return carry

    lax.fori_loop(0, _NCH_E, chunk, 0)
    plsc.subcore_barrier()
    pltpu.sync_copy(acc_sh.at[pl.ds(sid * _ROWS_PER_SUB, _ROWS_PER_SUB)],
                    out_hbm.at[cid, pl.ds(sid * _ROWS_PER_SUB,
                                          _ROWS_PER_SUB)])


def _make_sc_deg():
    return pl.kernel(
        _sc_deg_kernel,
        out_type=jax.ShapeDtypeStruct((2, _NP, _H), jnp.float32),
        mesh=_sc_mesh(),
        scratch_types=[
            pltpu.VMEM((_NCH_E + 1, _CH), jnp.int32),
            pltpu.VMEM((_CH, _H), jnp.float32),
            pltpu.VMEM_SHARED((_NP, _H), jnp.float32),
        ],
    )


def _sc_scatter_kernel(nch, table_hbm, src_hbm, dst_hbm, zeros_hbm, out_hbm,
                       idxs_v, idxd_v, gbuf0, gbuf1, sem0, sem1, acc_sh):
    cid = lax.axis_index("c")
    sid = lax.axis_index("s")
    w = cid * _NSUB + sid
    pltpu.sync_copy(src_hbm.at[w], idxs_v)
    pltpu.sync_copy(dst_hbm.at[w], idxd_v)
    pltpu.sync_copy(zeros_hbm.at[pl.ds(sid * _ROWS_PER_SUB, _ROWS_PER_SUB)],
                    acc_sh.at[pl.ds(sid * _ROWS_PER_SUB, _ROWS_PER_SUB)])
    plsc.subcore_barrier()

    pltpu.async_copy(table_hbm.at[idxs_v.at[0]], gbuf0, sem0)

    def pair(ch2, carry):
        ch = ch2 * 2
        pltpu.make_async_copy(table_hbm.at[idxs_v.at[ch]], gbuf0,
                              sem0).wait()
        pltpu.async_copy(table_hbm.at[idxs_v.at[ch + 1]], gbuf1, sem1)
        pltpu.sync_copy(gbuf0, acc_sh.at[idxd_v.at[ch]], add=True)
        pltpu.make_async_copy(table_hbm.at[idxs_v.at[ch + 1]], gbuf1,
                              sem1).wait()
        pltpu.async_copy(table_hbm.at[idxs_v.at[ch + 2]], gbuf0, sem0)
        pltpu.sync_copy(gbuf1, acc_sh.at[idxd_v.at[ch + 1]], add=True)
        return carry

    lax.fori_loop(0, nch // 2, pair, 0)
    pltpu.make_async_copy(table_hbm.at[idxs_v.at[nch]], gbuf0, sem0).wait()
    plsc.subcore_barrier()
    pltpu.sync_copy(acc_sh.at[pl.ds(sid * _ROWS_PER_SUB, _ROWS_PER_SUB)],
                    out_hbm.at[cid, pl.ds(sid * _ROWS_PER_SUB,
                                          _ROWS_PER_SUB)])


def _make_sc_scatter(nch):
    assert nch % 2 == 0
    return pl.kernel(
        functools.partial(_sc_scatter_kernel, nch),
        out_type=jax.ShapeDtypeStruct((2, _NP, _H), jnp.float32),
        mesh=_sc_mesh(),
        scratch_types=[
            pltpu.VMEM((nch + 1, _CH), jnp.int32),
            pltpu.VMEM((nch + 1, _CH), jnp.int32),
            pltpu.VMEM((_CH, _H), jnp.float32),
            pltpu.VMEM((_CH, _H), jnp.float32),
            pltpu.SemaphoreType.DMA,
            pltpu.SemaphoreType.DMA,
            pltpu.VMEM_SHARED((_NP, _H), jnp.float32),
        ],
    )


def _vm(n):
    return [pl.BlockSpec(memory_space=pltpu.VMEM)] * n


def _tc_call(body, n_in, out_shapes):
    return pl.pallas_call(
        body,
        in_specs=_vm(n_in),
        out_specs=[pl.BlockSpec(memory_space=pltpu.VMEM)] * len(out_shapes),
        out_shape=out_shapes,
    )


def _knn_call(sprefetch, hn, hnt3, brow, bcol3):
    grid_spec = pltpu.PrefetchScalarGridSpec(
        num_scalar_prefetch=1,
        grid=(_NRB,),
        in_specs=[
            pl.BlockSpec((_R, _H), lambda i, s: (i, 0)),
            pl.BlockSpec((_NCT, _H, _C), lambda i, s: (0, 0, 0)),
            pl.BlockSpec((_R, 1), lambda i, s: (i, 0)),
            pl.BlockSpec((_NCT, 1, _C), lambda i, s: (0, 0, 0)),
        ],
        out_specs=pl.BlockSpec((_R, _H), lambda i, s: (i, 0)),
    )
    return pl.pallas_call(
        _knn_body,
        grid_spec=grid_spec,
        out_shape=jax.ShapeDtypeStruct((_N, _H), jnp.int32),
    )(sprefetch, hn, hnt3, brow, bcol3)


def _pad_edges(e_arr, nch):
    npad = _NW * nch * _CH - e_arr.shape[0]
    filler = jnp.full((npad,), _NP - 1, jnp.int32)
    mat = jnp.concatenate([e_arr, filler]).reshape(_NW, nch, _CH)
    junk = jnp.full((_NW, 1, _CH), _NP - 1, jnp.int32)
    return jnp.concatenate([mat, junk], axis=1)


def kernel(x, edge_index, batch, W_emb, b_emb, Wc, bc, gn_w, gn_b, gn_ms,
           Wf, bf, fn_w, fn_b, fn_ms, Wg, bg):
    src = edge_index[0]
    dst = edge_index[1]
    brow = batch.reshape(_N, 1)
    bcol = batch.reshape(1, _N)

    dst3 = _pad_edges(dst, _NCH_E)
    eh = _E // 2
    srca3 = _pad_edges(src[:eh], _NCH_EH)
    dsta3 = _pad_edges(dst[:eh], _NCH_EH)
    srcb3 = _pad_edges(src[eh:], _NCH_EH)
    dstb3 = _pad_edges(dst[eh:], _NCH_EH)

    zeros128 = jnp.zeros((_NP, _H), jnp.float32)
    ones128 = jnp.ones((_CH, _H), jnp.float32)

    degp = _make_sc_deg()(dst3, ones128, zeros128)
    degp_sl = degp[:, :, :1]

    h0, hn, pm, mt = _tc_call(
        _prep_body, 5,
        [jax.ShapeDtypeStruct((_N, _H), jnp.float32),
         jax.ShapeDtypeStruct((_N, _H), jnp.float32),
         jax.ShapeDtypeStruct((_NG, _N), jnp.float32),
         jax.ShapeDtypeStruct((_N, _NG), jnp.float32)],
    )(x, W_emb, b_emb.reshape(1, _H), brow, bcol)

    bs = batch[0::_R]
    be = batch[_R - 1::_R]
    lo = jnp.searchsorted(batch, bs, side="left").astype(jnp.int32)
    hi = jnp.searchsorted(batch, be, side="right").astype(jnp.int32)
    sprefetch = jnp.stack([lo, hi])

    hnp = jnp.concatenate([hn, jnp.zeros((_NP - _N, _H), jnp.float32)])
    hnt3 = hnp.reshape(_NCT, _C, _H).transpose(0, 2, 1)
    bcolp = jnp.concatenate(
        [batch, jnp.full((_NP - _N,), -1, jnp.int32)]).reshape(_NCT, 1, _C)

    idx_wide = _knn_call(sprefetch, hn, hnt3, brow, bcolp)
    fsrc = idx_wide[:, :_K].reshape(-1)
    fsrc3 = _pad_edges(fsrc, _NCH_F)
    fdst = jnp.repeat(jnp.arange(_N, dtype=jnp.int32), _K)
    fdst3 = _pad_edges(fdst, _NCH_F)

    sc_edge = _make_sc_scatter(_NCH_EH)
    sc_feat = _make_sc_scatter(_NCH_F)

    wg1 = Wg[:_H]
    wg2 = Wg[_H:]

    h = h0
    pool = jnp.zeros((_NG, _H), jnp.float32)
    for i in range(_L - 1):
        u = _tc_call(
            _prescale_body, 3,
            [jax.ShapeDtypeStruct((_NP, _H), jnp.float32)],
        )(h, Wc[i], degp_sl)[0]

        acca = sc_edge(u, srca3, dsta3, zeros128)
        accb = sc_edge(u, srcb3, dstb3, zeros128)

        conv = _tc_call(
            _convsum_body, 5, [jax.ShapeDtypeStruct((_N, _H), jnp.float32)],
        )(acca, accb, u, degp_sl, bc[i].reshape(1, _H))[0]
        h1 = _tc_call(
            _gnorm_body, 6, [jax.ShapeDtypeStruct((_N, _H), jnp.float32)],
        )(conv, pm, mt, gn_w[i].reshape(1, _H), gn_b[i].reshape(1, _H),
          gn_ms[i].reshape(1, _H))[0]
        hwf = _tc_call(
            _matpad_body, 2, [jax.ShapeDtypeStruct((_NP, _H), jnp.float32)],
        )(h1, Wf[i])[0]

        accf = sc_feat(hwf, fsrc3, fdst3, zeros128)

        convf = _tc_call(
            _convf_body, 3, [jax.ShapeDtypeStruct((_N, _H), jnp.float32)],
        )(accf, hwf, bf[i].reshape(1, _H))[0]
        f = _tc_call(
            _gnorm_body, 6, [jax.ShapeDtypeStruct((_N, _H), jnp.float32)],
        )(convf, pm, mt, fn_w[i].reshape(1, _H), fn_b[i].reshape(1, _H),
          fn_ms[i].reshape(1, _H))[0]

        scale = 2.0 if i == _L - 2 else 1.0
        h, pool = _tc_call(
            _gate_body, 8,
            [jax.ShapeDtypeStruct((_N, _H), jnp.float32),
             jax.ShapeDtypeStruct((_NG, _H), jnp.float32)],
        )(h1, f, h, pm, wg1, wg2, bg.reshape(1, _H), pool / scale)
        pool = pool * scale

    return pool


# speedup vs baseline: 1.4729x; 1.4729x over previous
import functools

import jax
import jax.numpy as jnp
from jax import lax
from jax.experimental import pallas as pl
from jax.experimental.pallas import tpu as pltpu
from jax.experimental.pallas import tpu_sc as plsc

_N = 10000
_H = 128
_NG = 64
_K = 3
_E = 320000
_L = 3
_EPS = 1e-5
_NP = 10240
_R = 400
_C = 512
_NRB = _N // _R
_NCT = _NP // _C
_NW = 32
_NSUB = 16
_CH = 128
_NEG = float('-inf')

_HI = jax.lax.Precision.HIGHEST


def _edges_per_worker(e):
    per = -(-e // _NW)
    nch = -(-per // _CH)
    nch += nch % 2
    return nch, _NW * nch * _CH


_NCH_E, _EPAD = _edges_per_worker(_E)
_NCH_F, _FPAD = _edges_per_worker(_N * _K)


def _prep_body(x_ref, we_ref, be_ref, brow_ref, bcol_ref,
               h_ref, hn_ref, pm_ref, mt_ref):
    x = x_ref[...]
    h = jnp.dot(x, we_ref[...], preferred_element_type=jnp.float32,
                precision=jax.lax.Precision.DEFAULT) + be_ref[...]
    h_ref[...] = h
    nrm = jnp.sqrt(jnp.sum(h * h, axis=1, keepdims=True))
    hn_ref[...] = h / jnp.maximum(nrm, 1e-12)
    gids = lax.broadcasted_iota(jnp.int32, (_NG, _N), 0)
    m = (gids == bcol_ref[...]).astype(jnp.float32)
    cnt = jnp.sum(m, axis=1, keepdims=True)
    pm_ref[...] = m / jnp.maximum(cnt, 1.0)
    gidsr = lax.broadcasted_iota(jnp.int32, (_N, _NG), 1)
    mt_ref[...] = (gidsr == brow_ref[...]).astype(jnp.float32)


def _knn_body(s_ref, hn_ref, hnt_ref, brow_ref, bcol_ref, out_ref):
    rb = pl.program_id(0)
    lo = s_ref[0, rb]
    hi = s_ref[1, rb]
    c0 = lo // _C
    c1 = (hi + _C - 1) // _C
    rows = hn_ref[...]
    rbatch = brow_ref[...]
    bv = jnp.full((_R, _K), _NEG, jnp.float32)
    bi = lax.broadcasted_iota(jnp.int32, (_R, _K), 1)

    def tilestep(c, carry):
        bv, bi = carry
        base = c * _C
        cols = hnt_ref[c]
        sim = jnp.dot(rows, cols, preferred_element_type=jnp.float32,
                      precision=jax.lax.Precision.DEFAULT)
        cbatch = bcol_ref[c]
        sim = jnp.where(rbatch == cbatch, sim, _NEG)
        iot = lax.broadcasted_iota(jnp.int32, (_R, _C), 1) + base
        cv, ci = [], []
        for _ in range(_K):
            v = jnp.max(sim, axis=1, keepdims=True)
            sel = jnp.min(jnp.where(sim == v, iot, _NP), axis=1,
                          keepdims=True)
            cv.append(v)
            ci.append(sel)
            sim = jnp.where(iot == sel, _NEG, sim)
        allv = jnp.concatenate([bv] + cv, axis=1)
        alli = jnp.concatenate([bi] + ci, axis=1)
        nv, ni = [], []
        for _ in range(_K):
            v = jnp.max(allv, axis=1, keepdims=True)
            sel = jnp.min(jnp.where(allv == v, alli, _NP), axis=1,
                          keepdims=True)
            nv.append(v)
            ni.append(sel)
            hitm = alli == sel
            allv = jnp.where(hitm, _NEG, allv)
            alli = jnp.where(hitm, _NP, alli)
        return jnp.concatenate(nv, axis=1), jnp.concatenate(ni, axis=1)

    bv, bi = lax.fori_loop(c0, c1, tilestep, (bv, bi))
    out_ref[...] = jnp.concatenate(
        [bi, jnp.zeros((_R, _H - _K), jnp.int32)], axis=1)


def _dinv_from_deg(degp_ref):
    d = 1.0 + degp_ref[0][:_N] + degp_ref[1][:_N]
    return lax.rsqrt(d)


def _prescale_body(h_ref, w_ref, degp_ref, u_ref):
    hw = jnp.dot(h_ref[...], w_ref[...], preferred_element_type=jnp.float32,
                 precision=_HI)
    u = _dinv_from_deg(degp_ref) * hw
    u_ref[...] = jnp.concatenate(
        [u, jnp.zeros((_NP - _N, _H), jnp.float32)], axis=0)


def _graph_norm(h, pm_ref, mt_ref, w, b, ms):
    mean = jnp.dot(pm_ref[...], h, preferred_element_type=jnp.float32,
                   precision=_HI)
    out = h - jnp.dot(mt_ref[...], mean, preferred_element_type=jnp.float32,
                      precision=_HI) * ms
    var = jnp.dot(pm_ref[...], out * out, preferred_element_type=jnp.float32,
                  precision=_HI)
    inv = lax.rsqrt(var + _EPS)
    return w * out * jnp.dot(mt_ref[...], inv,
                             preferred_element_type=jnp.float32,
                             precision=_HI) + b


def _leaky(x):
    return jnp.where(x >= 0, x, 0.01 * x)


def _convsum_body(acc_ref, u_ref, degp_ref, bc_ref, conv_ref):
    s = (acc_ref[0][: _N] + acc_ref[1][: _N] + u_ref[: _N])
    conv_ref[...] = _dinv_from_deg(degp_ref) * s + bc_ref[...]


def _convf_body(accf_ref, hwf_ref, bf_ref, convf_ref):
    convf_ref[...] = 0.25 * (accf_ref[0][: _N] + accf_ref[1][: _N]
                             + hwf_ref[: _N]) + bf_ref[...]


def _gnorm_body(conv_ref, pm_ref, mt_ref, w_ref, b_ref, ms_ref, out_ref):
    out_ref[...] = _leaky(_graph_norm(conv_ref[...], pm_ref, mt_ref,
                                      w_ref[...], b_ref[...], ms_ref[...]))


def _matpad_body(h1_ref, wf_ref, hwf_ref):
    hwf = jnp.dot(h1_ref[...], wf_ref[...],
                  preferred_element_type=jnp.float32, precision=_HI)
    hwf_ref[...] = jnp.concatenate(
        [hwf, jnp.zeros((_NP - _N, _H), jnp.float32)], axis=0)


def _gate_body(h1_ref, f_ref, prev_ref, pm_ref, wg1_ref, wg2_ref, bg_ref,
               pool_in_ref, h_ref, pool_ref):
    h1 = h1_ref[...]
    f = f_ref[...]
    z = (jnp.dot(h1, wg1_ref[...], preferred_element_type=jnp.float32,
                 precision=_HI)
         + jnp.dot(f, wg2_ref[...], preferred_element_type=jnp.float32,
                   precision=_HI) + bg_ref[...])
    gate = 1.0 / (1.0 + jnp.exp(-z))
    h = gate * h1 + (1.0 - gate) * f + prev_ref[...]
    h_ref[...] = h
    pool = jnp.dot(pm_ref[...], h, preferred_element_type=jnp.float32,
                   precision=_HI)
    pool_ref[...] = pool_in_ref[...] + pool


def _sc_mesh():
    return plsc.VectorSubcoreMesh(core_axis_name="c", subcore_axis_name="s",
                                  num_cores=2, num_subcores=_NSUB)


_ROWS_PER_SUB = _NP // _NSUB


def _sc_deg_kernel(dst_hbm, ones_hbm, zeros_hbm, out_hbm, idx_v, ones_v,
                   acc_sh):
    cid = lax.axis_index("c")
    sid = lax.axis_index("s")
    w = cid * _NSUB + sid
    pltpu.sync_copy(dst_hbm.at[w], idx_v)
    pltpu.sync_copy(ones_hbm, ones_v)
    pltpu.sync_copy(zeros_hbm.at[pl.ds(sid * _ROWS_PER_SUB, _ROWS_PER_SUB)],
                    acc_sh.at[pl.ds(sid * _ROWS_PER_SUB, _ROWS_PER_SUB)])
    plsc.subcore_barrier()

    def chunk(ch, carry):
        pltpu.sync_copy(ones_v, acc_sh.at[idx_v.at[ch]], add=True)
        return carry

    lax.fori_loop(0, _NCH_E, chunk, 0)
    plsc.subcore_barrier()
    pltpu.sync_copy(acc_sh.at[pl.ds(sid * _ROWS_PER_SUB, _ROWS_PER_SUB)],
                    out_hbm.at[cid, pl.ds(sid * _ROWS_PER_SUB,
                                          _ROWS_PER_SUB)])


def _make_sc_deg():
    return pl.kernel(
        _sc_deg_kernel,
        out_type=jax.ShapeDtypeStruct((2, _NP, _H), jnp.float32),
        mesh=_sc_mesh(),
        scratch_types=[
            pltpu.VMEM((_NCH_E + 1, _CH), jnp.int32),
            pltpu.VMEM((_CH, _H), jnp.float32),
            pltpu.VMEM_SHARED((_NP, _H), jnp.float32),
        ],
    )


def _sc_scatter_kernel(nch, table_hbm, src_hbm, dst_hbm, zeros_hbm, out_hbm,
                       idxs_v, idxd_v, gbuf0, sem0, acc_sh):
    cid = lax.axis_index("c")
    sid = lax.axis_index("s")
    w = cid * _NSUB + sid
    pltpu.sync_copy(src_hbm.at[w], idxs_v)
    pltpu.sync_copy(dst_hbm.at[w], idxd_v)
    pltpu.sync_copy(zeros_hbm.at[pl.ds(sid * _ROWS_PER_SUB, _ROWS_PER_SUB)],
                    acc_sh.at[pl.ds(sid * _ROWS_PER_SUB, _ROWS_PER_SUB)])
    plsc.subcore_barrier()

    def chunk(ch, carry):
        pltpu.async_copy(table_hbm.at[idxs_v.at[ch]], gbuf0, sem0).wait()
        pltpu.sync_copy(gbuf0, acc_sh.at[idxd_v.at[ch]], add=True)
        return carry

    lax.fori_loop(0, nch, chunk, 0)
    plsc.subcore_barrier()
    pltpu.sync_copy(acc_sh.at[pl.ds(sid * _ROWS_PER_SUB, _ROWS_PER_SUB)],
                    out_hbm.at[cid, pl.ds(sid * _ROWS_PER_SUB,
                                          _ROWS_PER_SUB)])


def _make_sc_scatter(nch):
    return pl.kernel(
        functools.partial(_sc_scatter_kernel, nch),
        out_type=jax.ShapeDtypeStruct((2, _NP, _H), jnp.float32),
        mesh=_sc_mesh(),
        scratch_types=[
            pltpu.VMEM((nch + 1, _CH), jnp.int32),
            pltpu.VMEM((nch + 1, _CH), jnp.int32),
            pltpu.VMEM((_CH, _H), jnp.float32),
            pltpu.SemaphoreType.DMA,
            pltpu.VMEM_SHARED((_NP, _H), jnp.float32),
        ],
    )


def _vm(n):
    return [pl.BlockSpec(memory_space=pltpu.VMEM)] * n


def _tc_call(body, n_in, out_shapes):
    return pl.pallas_call(
        body,
        in_specs=_vm(n_in),
        out_specs=[pl.BlockSpec(memory_space=pltpu.VMEM)] * len(out_shapes),
        out_shape=out_shapes,
    )


def _knn_call(sprefetch, hn, hnt3, brow, bcol3):
    grid_spec = pltpu.PrefetchScalarGridSpec(
        num_scalar_prefetch=1,
        grid=(_NRB,),
        in_specs=[
            pl.BlockSpec((_R, _H), lambda i, s: (i, 0)),
            pl.BlockSpec((_NCT, _H, _C), lambda i, s: (0, 0, 0)),
            pl.BlockSpec((_R, 1), lambda i, s: (i, 0)),
            pl.BlockSpec((_NCT, 1, _C), lambda i, s: (0, 0, 0)),
        ],
        out_specs=pl.BlockSpec((_R, _H), lambda i, s: (i, 0)),
    )
    return pl.pallas_call(
        _knn_body,
        grid_spec=grid_spec,
        out_shape=jax.ShapeDtypeStruct((_N, _H), jnp.int32),
    )(sprefetch, hn, hnt3, brow, bcol3)


def _pad_edges(e_arr, nch):
    npad = _NW * nch * _CH - e_arr.shape[0]
    filler = jnp.full((npad,), _NP - 1, jnp.int32)
    mat = jnp.concatenate([e_arr, filler]).reshape(_NW, nch, _CH)
    junk = jnp.full((_NW, 1, _CH), _NP - 1, jnp.int32)
    return jnp.concatenate([mat, junk], axis=1)


def kernel(x, edge_index, batch, W_emb, b_emb, Wc, bc, gn_w, gn_b, gn_ms,
           Wf, bf, fn_w, fn_b, fn_ms, Wg, bg):
    src = edge_index[0]
    dst = edge_index[1]
    brow = batch.reshape(_N, 1)
    bcol = batch.reshape(1, _N)

    dst3 = _pad_edges(dst, _NCH_E)
    src3 = _pad_edges(src, _NCH_E)

    zeros128 = jnp.zeros((_NP, _H), jnp.float32)
    ones128 = jnp.ones((_CH, _H), jnp.float32)

    degp = _make_sc_deg()(dst3, ones128, zeros128)
    degp_sl = degp[:, :, :1]

    h0, hn, pm, mt = _tc_call(
        _prep_body, 5,
        [jax.ShapeDtypeStruct((_N, _H), jnp.float32),
         jax.ShapeDtypeStruct((_N, _H), jnp.float32),
         jax.ShapeDtypeStruct((_NG, _N), jnp.float32),
         jax.ShapeDtypeStruct((_N, _NG), jnp.float32)],
    )(x, W_emb, b_emb.reshape(1, _H), brow, bcol)

    bs = batch[0::_R]
    be = batch[_R - 1::_R]
    lo = jnp.searchsorted(batch, bs, side="left").astype(jnp.int32)
    hi = jnp.searchsorted(batch, be, side="right").astype(jnp.int32)
    sprefetch = jnp.stack([lo, hi])

    hnp = jnp.concatenate([hn, jnp.zeros((_NP - _N, _H), jnp.float32)])
    hnt3 = hnp.reshape(_NCT, _C, _H).transpose(0, 2, 1)
    bcolp = jnp.concatenate(
        [batch, jnp.full((_NP - _N,), -1, jnp.int32)]).reshape(_NCT, 1, _C)

    idx_wide = _knn_call(sprefetch, hn, hnt3, brow, bcolp)
    fsrc = idx_wide[:, :_K].reshape(-1)
    fsrc3 = _pad_edges(fsrc, _NCH_F)
    fdst = jnp.repeat(jnp.arange(_N, dtype=jnp.int32), _K)
    fdst3 = _pad_edges(fdst, _NCH_F)

    sc_edge = _make_sc_scatter(_NCH_E)
    sc_feat = _make_sc_scatter(_NCH_F)

    wg1 = Wg[:_H]
    wg2 = Wg[_H:]

    h = h0
    pool = jnp.zeros((_NG, _H), jnp.float32)
    for i in range(_L - 1):
        u = _tc_call(
            _prescale_body, 3,
            [jax.ShapeDtypeStruct((_NP, _H), jnp.float32)],
        )(h, Wc[i], degp_sl)[0]

        acc = sc_edge(u, src3, dst3, zeros128)

        conv = _tc_call(
            _convsum_body, 4, [jax.ShapeDtypeStruct((_N, _H), jnp.float32)],
        )(acc, u, degp_sl, bc[i].reshape(1, _H))[0]
        h1 = _tc_call(
            _gnorm_body, 6, [jax.ShapeDtypeStruct((_N, _H), jnp.float32)],
        )(conv, pm, mt, gn_w[i].reshape(1, _H), gn_b[i].reshape(1, _H),
          gn_ms[i].reshape(1, _H))[0]
        hwf = _tc_call(
            _matpad_body, 2, [jax.ShapeDtypeStruct((_NP, _H), jnp.float32)],
        )(h1, Wf[i])[0]

        accf = sc_feat(hwf, fsrc3, fdst3, zeros128)

        convf = _tc_call(
            _convf_body, 3, [jax.ShapeDtypeStruct((_N, _H), jnp.float32)],
        )(accf, hwf, bf[i].reshape(1, _H))[0]
        f = _tc_call(
            _gnorm_body, 6, [jax.ShapeDtypeStruct((_N, _H), jnp.float32)],
        )(convf, pm, mt, fn_w[i].reshape(1, _H), fn_b[i].reshape(1, _H),
          fn_ms[i].reshape(1, _H))[0]

        scale = 2.0 if i == _L - 2 else 1.0
        h, pool = _tc_call(
            _gate_body, 8,
            [jax.ShapeDtypeStruct((_N, _H), jnp.float32),
             jax.ShapeDtypeStruct((_NG, _H), jnp.float32)],
        )(h1, f, h, pm, wg1, wg2, bg.reshape(1, _H), pool / scale)
        pool = pool * scale

    return pool


# speedup vs baseline: 3.3071x; 2.2453x over previous
import functools

import jax
import jax.numpy as jnp
from jax import lax
from jax.experimental import pallas as pl
from jax.experimental.pallas import tpu as pltpu
from jax.experimental.pallas import tpu_sc as plsc

_N = 10000
_H = 128
_NG = 64
_K = 3
_E = 320000
_L = 3
_EPS = 1e-5
_NP = 10240
_R = 400
_C = 512
_NRB = _N // _R
_NCT = _NP // _C
_NW = 32
_NSUB = 16
_CH = 128
_NEG = float('-inf')

_HI = jax.lax.Precision.HIGHEST


def _edges_per_worker(e):
    per = -(-e // _NW)
    nch = -(-per // _CH)
    return nch, _NW * nch * _CH


_NCH_E, _EPAD = _edges_per_worker(_E)
_NCH_F, _FPAD = _edges_per_worker(_N * _K)


def _prep_body(x_ref, we_ref, be_ref, brow_ref, bcol_ref,
               h_ref, hn_ref, pm_ref, mt_ref):
    x = x_ref[...]
    h = jnp.dot(x, we_ref[...], preferred_element_type=jnp.float32,
                precision=jax.lax.Precision.DEFAULT) + be_ref[...]
    h_ref[...] = h
    nrm = jnp.sqrt(jnp.sum(h * h, axis=1, keepdims=True))
    hn_ref[...] = h / jnp.maximum(nrm, 1e-12)
    gids = lax.broadcasted_iota(jnp.int32, (_NG, _N), 0)
    m = (gids == bcol_ref[...]).astype(jnp.float32)
    cnt = jnp.sum(m, axis=1, keepdims=True)
    pm_ref[...] = m / jnp.maximum(cnt, 1.0)
    gidsr = lax.broadcasted_iota(jnp.int32, (_N, _NG), 1)
    mt_ref[...] = (gidsr == brow_ref[...]).astype(jnp.float32)


def _knn_body(s_ref, hn_ref, hnt_ref, brow_ref, bcol_ref, out_ref):
    rb = pl.program_id(0)
    lo = s_ref[0, rb]
    hi = s_ref[1, rb]
    c0 = lo // _C
    c1 = (hi + _C - 1) // _C
    rows = hn_ref[...]
    rbatch = brow_ref[...]
    bv = jnp.full((_R, _K), _NEG, jnp.float32)
    bi = lax.broadcasted_iota(jnp.int32, (_R, _K), 1)

    def tilestep(c, carry):
        bv, bi = carry
        base = c * _C
        cols = hnt_ref[c]
        sim = jnp.dot(rows, cols, preferred_element_type=jnp.float32,
                      precision=jax.lax.Precision.DEFAULT)
        cbatch = bcol_ref[c]
        sim = jnp.where(rbatch == cbatch, sim, _NEG)
        iot = lax.broadcasted_iota(jnp.int32, (_R, _C), 1) + base
        cv, ci = [], []
        for _ in range(_K):
            v = jnp.max(sim, axis=1, keepdims=True)
            sel = jnp.min(jnp.where(sim == v, iot, _NP), axis=1,
                          keepdims=True)
            cv.append(v)
            ci.append(sel)
            sim = jnp.where(iot == sel, _NEG, sim)
        allv = jnp.concatenate([bv] + cv, axis=1)
        alli = jnp.concatenate([bi] + ci, axis=1)
        nv, ni = [], []
        for _ in range(_K):
            v = jnp.max(allv, axis=1, keepdims=True)
            sel = jnp.min(jnp.where(allv == v, alli, _NP), axis=1,
                          keepdims=True)
            nv.append(v)
            ni.append(sel)
            hitm = alli == sel
            allv = jnp.where(hitm, _NEG, allv)
            alli = jnp.where(hitm, _NP, alli)
        return jnp.concatenate(nv, axis=1), jnp.concatenate(ni, axis=1)

    bv, bi = lax.fori_loop(c0, c1, tilestep, (bv, bi))
    out_ref[...] = jnp.concatenate(
        [bi, jnp.zeros((_R, _H - _K), jnp.int32)], axis=1)


def _dinv_from_deg(degp_ref):
    d = 1.0 + degp_ref[0][:_N] + degp_ref[1][:_N]
    return lax.rsqrt(d)


def _prescale_body(h_ref, w_ref, degp_ref, u_ref):
    hw = jnp.dot(h_ref[...], w_ref[...], preferred_element_type=jnp.float32,
                 precision=_HI)
    u = _dinv_from_deg(degp_ref) * hw
    u_ref[...] = jnp.concatenate(
        [u, jnp.zeros((_NP - _N, _H), jnp.float32)], axis=0)


def _graph_norm(h, pm_ref, mt_ref, w, b, ms):
    mean = jnp.dot(pm_ref[...], h, preferred_element_type=jnp.float32,
                   precision=_HI)
    out = h - jnp.dot(mt_ref[...], mean, preferred_element_type=jnp.float32,
                      precision=_HI) * ms
    var = jnp.dot(pm_ref[...], out * out, preferred_element_type=jnp.float32,
                  precision=_HI)
    inv = lax.rsqrt(var + _EPS)
    return w * out * jnp.dot(mt_ref[...], inv,
                             preferred_element_type=jnp.float32,
                             precision=_HI) + b


def _leaky(x):
    return jnp.where(x >= 0, x, 0.01 * x)


def _convsum_body(acc_ref, u_ref, degp_ref, bc_ref, conv_ref):
    s = (acc_ref[0][: _N] + acc_ref[1][: _N] + u_ref[: _N])
    conv_ref[...] = _dinv_from_deg(degp_ref) * s + bc_ref[...]


def _convf_body(accf_ref, hwf_ref, bf_ref, convf_ref):
    convf_ref[...] = 0.25 * (accf_ref[0][: _N] + accf_ref[1][: _N]
                             + hwf_ref[: _N]) + bf_ref[...]


def _gnorm_body(conv_ref, pm_ref, mt_ref, w_ref, b_ref, ms_ref, out_ref):
    out_ref[...] = _leaky(_graph_norm(conv_ref[...], pm_ref, mt_ref,
                                      w_ref[...], b_ref[...], ms_ref[...]))


def _matpad_body(h1_ref, wf_ref, hwf_ref):
    hwf = jnp.dot(h1_ref[...], wf_ref[...],
                  preferred_element_type=jnp.float32, precision=_HI)
    hwf_ref[...] = jnp.concatenate(
        [hwf, jnp.zeros((_NP - _N, _H), jnp.float32)], axis=0)


def _gate_body(h1_ref, f_ref, prev_ref, pm_ref, wg1_ref, wg2_ref, bg_ref,
               pool_in_ref, h_ref, pool_ref):
    h1 = h1_ref[...]
    f = f_ref[...]
    z = (jnp.dot(h1, wg1_ref[...], preferred_element_type=jnp.float32,
                 precision=_HI)
         + jnp.dot(f, wg2_ref[...], preferred_element_type=jnp.float32,
                   precision=_HI) + bg_ref[...])
    gate = 1.0 / (1.0 + jnp.exp(-z))
    h = gate * h1 + (1.0 - gate) * f + prev_ref[...]
    h_ref[...] = h
    pool = jnp.dot(pm_ref[...], h, preferred_element_type=jnp.float32,
                   precision=_HI)
    pool_ref[...] = pool_in_ref[...] + pool


def _sc_mesh():
    return plsc.VectorSubcoreMesh(core_axis_name="c", subcore_axis_name="s",
                                  num_cores=2, num_subcores=_NSUB)


_ROWS_PER_SUB = _NP // _NSUB


def _sc_deg_kernel(dst_hbm, ones_hbm, zeros_hbm, out_hbm, idx_v, ones_v,
                   acc_sh):
    cid = lax.axis_index("c")
    sid = lax.axis_index("s")
    w = cid * _NSUB + sid
    pltpu.sync_copy(dst_hbm.at[w], idx_v)
    pltpu.sync_copy(ones_hbm, ones_v)
    pltpu.sync_copy(zeros_hbm.at[pl.ds(sid * _ROWS_PER_SUB, _ROWS_PER_SUB)],
                    acc_sh.at[pl.ds(sid * _ROWS_PER_SUB, _ROWS_PER_SUB)])
    plsc.subcore_barrier()

    def chunk(ch, carry):
        pltpu.sync_copy(ones_v, acc_sh.at[idx_v.at[ch]], add=True)
        return carry

    lax.fori_loop(0, _NCH_E, chunk, 0)
    plsc.subcore_barrier()
    pltpu.sync_copy(acc_sh.at[pl.ds(sid * _ROWS_PER_SUB, _ROWS_PER_SUB)],
                    out_hbm.at[cid, pl.ds(sid * _ROWS_PER_SUB,
                                          _ROWS_PER_SUB)])


def _make_sc_deg():
    return pl.kernel(
        _sc_deg_kernel,
        out_type=jax.ShapeDtypeStruct((2, _NP, _H), jnp.float32),
        mesh=_sc_mesh(),
        scratch_types=[
            pltpu.VMEM((_NCH_E + 1, _CH), jnp.int32),
            pltpu.VMEM((_CH, _H), jnp.float32),
            pltpu.VMEM_SHARED((_NP, _H), jnp.float32),
        ],
    )


def _sc_scatter_kernel(nch, table_hbm, src_hbm, dst_hbm, zeros_hbm, out_hbm,
                       idxs_v, idxd_v, gbuf0, sem0, acc_sh):
    cid = lax.axis_index("c")
    sid = lax.axis_index("s")
    w = cid * _NSUB + sid
    pltpu.sync_copy(src_hbm.at[w], idxs_v)
    pltpu.sync_copy(dst_hbm.at[w], idxd_v)
    pltpu.sync_copy(zeros_hbm.at[pl.ds(sid * _ROWS_PER_SUB, _ROWS_PER_SUB)],
                    acc_sh.at[pl.ds(sid * _ROWS_PER_SUB, _ROWS_PER_SUB)])
    plsc.subcore_barrier()

    def chunk(ch, carry):
        pltpu.async_copy(table_hbm.at[idxs_v.at[ch]], gbuf0, sem0).wait()
        pltpu.sync_copy(gbuf0, acc_sh.at[idxd_v.at[ch]], add=True)
        return carry

    lax.fori_loop(0, nch, chunk, 0)
    plsc.subcore_barrier()
    pltpu.sync_copy(acc_sh.at[pl.ds(sid * _ROWS_PER_SUB, _ROWS_PER_SUB)],
                    out_hbm.at[cid, pl.ds(sid * _ROWS_PER_SUB,
                                          _ROWS_PER_SUB)])


def _make_sc_scatter(nch):
    return pl.kernel(
        functools.partial(_sc_scatter_kernel, nch),
        out_type=jax.ShapeDtypeStruct((2, _NP, _H), jnp.float32),
        mesh=_sc_mesh(),
        scratch_types=[
            pltpu.VMEM((nch + 1, _CH), jnp.int32),
            pltpu.VMEM((nch + 1, _CH), jnp.int32),
            pltpu.VMEM((_CH, _H), jnp.float32),
            pltpu.SemaphoreType.DMA,
            pltpu.VMEM_SHARED((_NP, _H), jnp.float32),
        ],
    )


def _vm(n):
    return [pl.BlockSpec(memory_space=pltpu.VMEM)] * n


def _tc_call(body, n_in, out_shapes):
    return pl.pallas_call(
        body,
        in_specs=_vm(n_in),
        out_specs=[pl.BlockSpec(memory_space=pltpu.VMEM)] * len(out_shapes),
        out_shape=out_shapes,
    )


def _knn_call(sprefetch, hn, hnt3, brow, bcol3):
    grid_spec = pltpu.PrefetchScalarGridSpec(
        num_scalar_prefetch=1,
        grid=(_NRB,),
        in_specs=[
            pl.BlockSpec((_R, _H), lambda i, s: (i, 0)),
            pl.BlockSpec((_NCT, _H, _C), lambda i, s: (0, 0, 0)),
            pl.BlockSpec((_R, 1), lambda i, s: (i, 0)),
            pl.BlockSpec((_NCT, 1, _C), lambda i, s: (0, 0, 0)),
        ],
        out_specs=pl.BlockSpec((_R, _H), lambda i, s: (i, 0)),
    )
    return pl.pallas_call(
        _knn_body,
        grid_spec=grid_spec,
        out_shape=jax.ShapeDtypeStruct((_N, _H), jnp.int32),
    )(sprefetch, hn, hnt3, brow, bcol3)


def _pad_edges(e_arr, nch):
    npad = _NW * nch * _CH - e_arr.shape[0]
    filler = _N + (jnp.arange(npad, dtype=jnp.int32) % (_NP - _N))
    mat = jnp.concatenate([e_arr, filler]).reshape(_NW, nch, _CH)
    junk = jnp.full((_NW, 1, _CH), _N, jnp.int32)
    return jnp.concatenate([mat, junk], axis=1)


def kernel(x, edge_index, batch, W_emb, b_emb, Wc, bc, gn_w, gn_b, gn_ms,
           Wf, bf, fn_w, fn_b, fn_ms, Wg, bg):
    src = edge_index[0]
    dst = edge_index[1]
    brow = batch.reshape(_N, 1)
    bcol = batch.reshape(1, _N)

    dst3 = _pad_edges(dst, _NCH_E)
    src3 = _pad_edges(src, _NCH_E)

    zeros128 = jnp.zeros((_NP, _H), jnp.float32)
    ones128 = jnp.ones((_CH, _H), jnp.float32)

    degp = _make_sc_deg()(dst3, ones128, zeros128)
    degp_sl = degp[:, :, :1]

    h0, hn, pm, mt = _tc_call(
        _prep_body, 5,
        [jax.ShapeDtypeStruct((_N, _H), jnp.float32),
         jax.ShapeDtypeStruct((_N, _H), jnp.float32),
         jax.ShapeDtypeStruct((_NG, _N), jnp.float32),
         jax.ShapeDtypeStruct((_N, _NG), jnp.float32)],
    )(x, W_emb, b_emb.reshape(1, _H), brow, bcol)

    bs = batch[0::_R]
    be = batch[_R - 1::_R]
    lo = jnp.searchsorted(batch, bs, side="left").astype(jnp.int32)
    hi = jnp.searchsorted(batch, be, side="right").astype(jnp.int32)
    sprefetch = jnp.stack([lo, hi])

    hnp = jnp.concatenate([hn, jnp.zeros((_NP - _N, _H), jnp.float32)])
    hnt3 = hnp.reshape(_NCT, _C, _H).transpose(0, 2, 1)
    bcolp = jnp.concatenate(
        [batch, jnp.full((_NP - _N,), -1, jnp.int32)]).reshape(_NCT, 1, _C)

    idx_wide = _knn_call(sprefetch, hn, hnt3, brow, bcolp)
    fsrc = idx_wide[:, :_K].reshape(-1)
    fsrc3 = _pad_edges(fsrc, _NCH_F)
    fdst = jnp.repeat(jnp.arange(_N, dtype=jnp.int32), _K)
    fdst3 = _pad_edges(fdst, _NCH_F)

    sc_edge = _make_sc_scatter(_NCH_E)
    sc_feat = _make_sc_scatter(_NCH_F)

    wg1 = Wg[:_H]
    wg2 = Wg[_H:]

    h = h0
    pool = jnp.zeros((_NG, _H), jnp.float32)
    for i in range(_L - 1):
        u = _tc_call(
            _prescale_body, 3,
            [jax.ShapeDtypeStruct((_NP, _H), jnp.float32)],
        )(h, Wc[i], degp_sl)[0]

        acc = sc_edge(u, src3, dst3, zeros128)

        conv = _tc_call(
            _convsum_body, 4, [jax.ShapeDtypeStruct((_N, _H), jnp.float32)],
        )(acc, u, degp_sl, bc[i].reshape(1, _H))[0]
        h1 = _tc_call(
            _gnorm_body, 6, [jax.ShapeDtypeStruct((_N, _H), jnp.float32)],
        )(conv, pm, mt, gn_w[i].reshape(1, _H), gn_b[i].reshape(1, _H),
          gn_ms[i].reshape(1, _H))[0]
        hwf = _tc_call(
            _matpad_body, 2, [jax.ShapeDtypeStruct((_NP, _H), jnp.float32)],
        )(h1, Wf[i])[0]

        accf = sc_feat(hwf, fsrc3, fdst3, zeros128)

        convf = _tc_call(
            _convf_body, 3, [jax.ShapeDtypeStruct((_N, _H), jnp.float32)],
        )(accf, hwf, bf[i].reshape(1, _H))[0]
        f = _tc_call(
            _gnorm_body, 6, [jax.ShapeDtypeStruct((_N, _H), jnp.float32)],
        )(convf, pm, mt, fn_w[i].reshape(1, _H), fn_b[i].reshape(1, _H),
          fn_ms[i].reshape(1, _H))[0]

        scale = 2.0 if i == _L - 2 else 1.0
        h, pool = _tc_call(
            _gate_body, 8,
            [jax.ShapeDtypeStruct((_N, _H), jnp.float32),
             jax.ShapeDtypeStruct((_NG, _H), jnp.float32)],
        )(h1, f, h, pm, wg1, wg2, bg.reshape(1, _H), pool / scale)
        pool = pool * scale

    return pool


# speedup vs baseline: 3.3193x; 1.0037x over previous
import functools

import jax
import jax.numpy as jnp
from jax import lax
from jax.experimental import pallas as pl
from jax.experimental.pallas import tpu as pltpu
from jax.experimental.pallas import tpu_sc as plsc

_N = 10000
_H = 128
_NG = 64
_K = 3
_E = 320000
_L = 3
_EPS = 1e-5
_NP = 10240
_R = 400
_C = 512
_NRB = _N // _R
_NCT = _NP // _C
_NW = 32
_NSUB = 16
_CH = 128
_NEG = float('-inf')

_HI = jax.lax.Precision.HIGHEST


def _edges_per_worker(e):
    per = -(-e // _NW)
    nch = -(-per // _CH)
    return nch, _NW * nch * _CH


_NCH_E, _EPAD = _edges_per_worker(_E)
_NCH_F, _FPAD = _edges_per_worker(_N * _K)


def _prep_body(x_ref, we_ref, be_ref, brow_ref, bcol_ref,
               h_ref, hn_ref, pm_ref, mt_ref):
    x = x_ref[...]
    h = jnp.dot(x, we_ref[...], preferred_element_type=jnp.float32,
                precision=jax.lax.Precision.DEFAULT) + be_ref[...]
    h_ref[...] = h
    nrm = jnp.sqrt(jnp.sum(h * h, axis=1, keepdims=True))
    hn_ref[...] = h / jnp.maximum(nrm, 1e-12)
    gids = lax.broadcasted_iota(jnp.int32, (_NG, _N), 0)
    m = (gids == bcol_ref[...]).astype(jnp.float32)
    cnt = jnp.sum(m, axis=1, keepdims=True)
    pm_ref[...] = m / jnp.maximum(cnt, 1.0)
    gidsr = lax.broadcasted_iota(jnp.int32, (_N, _NG), 1)
    mt_ref[...] = (gidsr == brow_ref[...]).astype(jnp.float32)


def _knn_body(s_ref, hn_ref, hnt_ref, brow_ref, bcol_ref, out_ref):
    rb = pl.program_id(0)
    lo = s_ref[0, rb]
    hi = s_ref[1, rb]
    c0 = lo // _C
    c1 = (hi + _C - 1) // _C
    rows = hn_ref[...]
    rbatch = brow_ref[...]
    bv = jnp.full((_R, _K), _NEG, jnp.float32)
    bi = lax.broadcasted_iota(jnp.int32, (_R, _K), 1)

    def tilestep(c, carry):
        bv, bi = carry
        base = c * _C
        cols = hnt_ref[c]
        sim = jnp.dot(rows, cols, preferred_element_type=jnp.float32,
                      precision=jax.lax.Precision.DEFAULT)
        cbatch = bcol_ref[c]
        sim = jnp.where(rbatch == cbatch, sim, _NEG)
        iot = lax.broadcasted_iota(jnp.int32, (_R, _C), 1) + base
        cv, ci = [], []
        for _ in range(_K):
            v = jnp.max(sim, axis=1, keepdims=True)
            sel = jnp.min(jnp.where(sim == v, iot, _NP), axis=1,
                          keepdims=True)
            cv.append(v)
            ci.append(sel)
            sim = jnp.where(iot == sel, _NEG, sim)
        allv = jnp.concatenate([bv] + cv, axis=1)
        alli = jnp.concatenate([bi] + ci, axis=1)
        nv, ni = [], []
        for _ in range(_K):
            v = jnp.max(allv, axis=1, keepdims=True)
            sel = jnp.min(jnp.where(allv == v, alli, _NP), axis=1,
                          keepdims=True)
            nv.append(v)
            ni.append(sel)
            hitm = alli == sel
            allv = jnp.where(hitm, _NEG, allv)
            alli = jnp.where(hitm, _NP, alli)
        return jnp.concatenate(nv, axis=1), jnp.concatenate(ni, axis=1)

    bv, bi = lax.fori_loop(c0, c1, tilestep, (bv, bi))
    out_ref[...] = jnp.concatenate(
        [bi, jnp.zeros((_R, _H - _K), jnp.int32)], axis=1)


def _dinv_from_deg(degp_ref):
    d = 1.0 + degp_ref[0][:_N] + degp_ref[1][:_N]
    return lax.rsqrt(d)


def _prescale_body(h_ref, w_ref, degp_ref, u_ref):
    hw = jnp.dot(h_ref[...], w_ref[...], preferred_element_type=jnp.float32,
                 precision=_HI)
    u = _dinv_from_deg(degp_ref) * hw
    u_ref[...] = jnp.concatenate(
        [u, jnp.zeros((_NP - _N, _H), jnp.float32)], axis=0)


def _graph_norm(h, pm_ref, mt_ref, w, b, ms):
    mean = jnp.dot(pm_ref[...], h, preferred_element_type=jnp.float32,
                   precision=_HI)
    out = h - jnp.dot(mt_ref[...], mean, preferred_element_type=jnp.float32,
                      precision=_HI) * ms
    var = jnp.dot(pm_ref[...], out * out, preferred_element_type=jnp.float32,
                  precision=_HI)
    inv = lax.rsqrt(var + _EPS)
    return w * out * jnp.dot(mt_ref[...], inv,
                             preferred_element_type=jnp.float32,
                             precision=_HI) + b


def _leaky(x):
    return jnp.where(x >= 0, x, 0.01 * x)


def _convsum_body(acc_ref, u_ref, degp_ref, bc_ref, conv_ref):
    s = (acc_ref[0][: _N] + acc_ref[1][: _N] + u_ref[: _N])
    conv_ref[...] = _dinv_from_deg(degp_ref) * s + bc_ref[...]


def _convf_body(accf_ref, hwf_ref, bf_ref, convf_ref):
    convf_ref[...] = 0.25 * (accf_ref[0][: _N] + accf_ref[1][: _N]
                             + hwf_ref[: _N]) + bf_ref[...]


def _gnorm_body(conv_ref, pm_ref, mt_ref, w_ref, b_ref, ms_ref, out_ref):
    out_ref[...] = _leaky(_graph_norm(conv_ref[...], pm_ref, mt_ref,
                                      w_ref[...], b_ref[...], ms_ref[...]))


def _gnorm_mat_body(conv_ref, pm_ref, mt_ref, w_ref, b_ref, ms_ref, wf_ref,
                    h1_ref, hwf_ref):
    h1 = _leaky(_graph_norm(conv_ref[...], pm_ref, mt_ref,
                            w_ref[...], b_ref[...], ms_ref[...]))
    h1_ref[...] = h1
    hwf = jnp.dot(h1, wf_ref[...], preferred_element_type=jnp.float32,
                  precision=_HI)
    hwf_ref[...] = jnp.concatenate(
        [hwf, jnp.zeros((_NP - _N, _H), jnp.float32)], axis=0)


def _convf_gnorm_body(accf_ref, hwf_ref, bf_ref, pm_ref, mt_ref,
                      w_ref, b_ref, ms_ref, f_ref):
    convf = 0.25 * (accf_ref[0][: _N] + accf_ref[1][: _N]
                    + hwf_ref[: _N]) + bf_ref[...]
    f_ref[...] = _leaky(_graph_norm(convf, pm_ref, mt_ref,
                                    w_ref[...], b_ref[...], ms_ref[...]))


def _matpad_body(h1_ref, wf_ref, hwf_ref):
    hwf = jnp.dot(h1_ref[...], wf_ref[...],
                  preferred_element_type=jnp.float32, precision=_HI)
    hwf_ref[...] = jnp.concatenate(
        [hwf, jnp.zeros((_NP - _N, _H), jnp.float32)], axis=0)


def _gate_body(h1_ref, f_ref, prev_ref, pm_ref, wg1_ref, wg2_ref, bg_ref,
               pool_in_ref, h_ref, pool_ref):
    h1 = h1_ref[...]
    f = f_ref[...]
    z = (jnp.dot(h1, wg1_ref[...], preferred_element_type=jnp.float32,
                 precision=_HI)
         + jnp.dot(f, wg2_ref[...], preferred_element_type=jnp.float32,
                   precision=_HI) + bg_ref[...])
    gate = 1.0 / (1.0 + jnp.exp(-z))
    h = gate * h1 + (1.0 - gate) * f + prev_ref[...]
    h_ref[...] = h
    pool = jnp.dot(pm_ref[...], h, preferred_element_type=jnp.float32,
                   precision=_HI)
    pool_ref[...] = pool_in_ref[...] + pool


def _sc_mesh():
    return plsc.VectorSubcoreMesh(core_axis_name="c", subcore_axis_name="s",
                                  num_cores=2, num_subcores=_NSUB)


_ROWS_PER_SUB = _NP // _NSUB


def _sc_deg_kernel(dst_hbm, ones_hbm, zeros_hbm, out_hbm, idx_v, ones_v,
                   acc_sh):
    cid = lax.axis_index("c")
    sid = lax.axis_index("s")
    w = cid * _NSUB + sid
    pltpu.sync_copy(dst_hbm.at[w], idx_v)
    pltpu.sync_copy(ones_hbm, ones_v)
    pltpu.sync_copy(zeros_hbm.at[pl.ds(sid * _ROWS_PER_SUB, _ROWS_PER_SUB)],
                    acc_sh.at[pl.ds(sid * _ROWS_PER_SUB, _ROWS_PER_SUB)])
    plsc.subcore_barrier()

    def chunk(ch, carry):
        pltpu.sync_copy(ones_v, acc_sh.at[idx_v.at[ch]], add=True)
        return carry

    lax.fori_loop(0, _NCH_E, chunk, 0)
    plsc.subcore_barrier()
    pltpu.sync_copy(acc_sh.at[pl.ds(sid * _ROWS_PER_SUB, _ROWS_PER_SUB)],
                    out_hbm.at[cid, pl.ds(sid * _ROWS_PER_SUB,
                                          _ROWS_PER_SUB)])


def _make_sc_deg():
    return pl.kernel(
        _sc_deg_kernel,
        out_type=jax.ShapeDtypeStruct((2, _NP, _H), jnp.float32),
        mesh=_sc_mesh(),
        scratch_types=[
            pltpu.VMEM((_NCH_E + 1, _CH), jnp.int32),
            pltpu.VMEM((_CH, _H), jnp.float32),
            pltpu.VMEM_SHARED((_NP, _H), jnp.float32),
        ],
    )


def _sc_scatter_kernel(nch, table_hbm, src_hbm, dst_hbm, zeros_hbm, out_hbm,
                       idxs_v, idxd_v, gbuf0, sem0, acc_sh):
    cid = lax.axis_index("c")
    sid = lax.axis_index("s")
    w = cid * _NSUB + sid
    pltpu.sync_copy(src_hbm.at[w], idxs_v)
    pltpu.sync_copy(dst_hbm.at[w], idxd_v)
    pltpu.sync_copy(zeros_hbm.at[pl.ds(sid * _ROWS_PER_SUB, _ROWS_PER_SUB)],
                    acc_sh.at[pl.ds(sid * _ROWS_PER_SUB, _ROWS_PER_SUB)])
    plsc.subcore_barrier()

    def chunk(ch, carry):
        pltpu.async_copy(table_hbm.at[idxs_v.at[ch]], gbuf0, sem0).wait()
        pltpu.sync_copy(gbuf0, acc_sh.at[idxd_v.at[ch]], add=True)
        return carry

    lax.fori_loop(0, nch, chunk, 0)
    plsc.subcore_barrier()
    pltpu.sync_copy(acc_sh.at[pl.ds(sid * _ROWS_PER_SUB, _ROWS_PER_SUB)],
                    out_hbm.at[cid, pl.ds(sid * _ROWS_PER_SUB,
                                          _ROWS_PER_SUB)])


def _make_sc_scatter(nch):
    return pl.kernel(
        functools.partial(_sc_scatter_kernel, nch),
        out_type=jax.ShapeDtypeStruct((2, _NP, _H), jnp.float32),
        mesh=_sc_mesh(),
        scratch_types=[
            pltpu.VMEM((nch + 1, _CH), jnp.int32),
            pltpu.VMEM((nch + 1, _CH), jnp.int32),
            pltpu.VMEM((_CH, _H), jnp.float32),
            pltpu.SemaphoreType.DMA,
            pltpu.VMEM_SHARED((_NP, _H), jnp.float32),
        ],
    )


def _vm(n):
    return [pl.BlockSpec(memory_space=pltpu.VMEM)] * n


def _tc_call(body, n_in, out_shapes):
    return pl.pallas_call(
        body,
        in_specs=_vm(n_in),
        out_specs=[pl.BlockSpec(memory_space=pltpu.VMEM)] * len(out_shapes),
        out_shape=out_shapes,
    )


def _knn_call(sprefetch, hn, hnt3, brow, bcol3):
    grid_spec = pltpu.PrefetchScalarGridSpec(
        num_scalar_prefetch=1,
        grid=(_NRB,),
        in_specs=[
            pl.BlockSpec((_R, _H), lambda i, s: (i, 0)),
            pl.BlockSpec((_NCT, _H, _C), lambda i, s: (0, 0, 0)),
            pl.BlockSpec((_R, 1), lambda i, s: (i, 0)),
            pl.BlockSpec((_NCT, 1, _C), lambda i, s: (0, 0, 0)),
        ],
        out_specs=pl.BlockSpec((_R, _H), lambda i, s: (i, 0)),
    )
    return pl.pallas_call(
        _knn_body,
        grid_spec=grid_spec,
        out_shape=jax.ShapeDtypeStruct((_N, _H), jnp.int32),
    )(sprefetch, hn, hnt3, brow, bcol3)


def _pad_edges(e_arr, nch):
    npad = _NW * nch * _CH - e_arr.shape[0]
    filler = _N + (jnp.arange(npad, dtype=jnp.int32) % (_NP - _N))
    mat = jnp.concatenate([e_arr, filler]).reshape(_NW, nch, _CH)
    junk = jnp.full((_NW, 1, _CH), _N, jnp.int32)
    return jnp.concatenate([mat, junk], axis=1)


def kernel(x, edge_index, batch, W_emb, b_emb, Wc, bc, gn_w, gn_b, gn_ms,
           Wf, bf, fn_w, fn_b, fn_ms, Wg, bg):
    src = edge_index[0]
    dst = edge_index[1]
    brow = batch.reshape(_N, 1)
    bcol = batch.reshape(1, _N)

    dst3 = _pad_edges(dst, _NCH_E)
    src3 = _pad_edges(src, _NCH_E)

    zeros128 = jnp.zeros((_NP, _H), jnp.float32)
    ones128 = jnp.ones((_CH, _H), jnp.float32)

    degp = _make_sc_deg()(dst3, ones128, zeros128)
    degp_sl = degp[:, :, :1]

    h0, hn, pm, mt = _tc_call(
        _prep_body, 5,
        [jax.ShapeDtypeStruct((_N, _H), jnp.float32),
         jax.ShapeDtypeStruct((_N, _H), jnp.float32),
         jax.ShapeDtypeStruct((_NG, _N), jnp.float32),
         jax.ShapeDtypeStruct((_N, _NG), jnp.float32)],
    )(x, W_emb, b_emb.reshape(1, _H), brow, bcol)

    bs = batch[0::_R]
    be = batch[_R - 1::_R]
    lo = jnp.searchsorted(batch, bs, side="left").astype(jnp.int32)
    hi = jnp.searchsorted(batch, be, side="right").astype(jnp.int32)
    sprefetch = jnp.stack([lo, hi])

    hnp = jnp.concatenate([hn, jnp.zeros((_NP - _N, _H), jnp.float32)])
    hnt3 = hnp.reshape(_NCT, _C, _H).transpose(0, 2, 1)
    bcolp = jnp.concatenate(
        [batch, jnp.full((_NP - _N,), -1, jnp.int32)]).reshape(_NCT, 1, _C)

    idx_wide = _knn_call(sprefetch, hn, hnt3, brow, bcolp)
    fsrc = idx_wide[:, :_K].reshape(-1)
    fsrc3 = _pad_edges(fsrc, _NCH_F)
    fdst = jnp.repeat(jnp.arange(_N, dtype=jnp.int32), _K)
    fdst3 = _pad_edges(fdst, _NCH_F)

    sc_edge = _make_sc_scatter(_NCH_E)
    sc_feat = _make_sc_scatter(_NCH_F)

    wg1 = Wg[:_H]
    wg2 = Wg[_H:]

    h = h0
    pool = jnp.zeros((_NG, _H), jnp.float32)
    for i in range(_L - 1):
        u = _tc_call(
            _prescale_body, 3,
            [jax.ShapeDtypeStruct((_NP, _H), jnp.float32)],
        )(h, Wc[i], degp_sl)[0]

        acc = sc_edge(u, src3, dst3, zeros128)

        conv = _tc_call(
            _convsum_body, 4, [jax.ShapeDtypeStruct((_N, _H), jnp.float32)],
        )(acc, u, degp_sl, bc[i].reshape(1, _H))[0]
        h1, hwf = _tc_call(
            _gnorm_mat_body, 7,
            [jax.ShapeDtypeStruct((_N, _H), jnp.float32),
             jax.ShapeDtypeStruct((_NP, _H), jnp.float32)],
        )(conv, pm, mt, gn_w[i].reshape(1, _H), gn_b[i].reshape(1, _H),
          gn_ms[i].reshape(1, _H), Wf[i])

        accf = sc_feat(hwf, fsrc3, fdst3, zeros128)

        convf = _tc_call(
            _convf_body, 3, [jax.ShapeDtypeStruct((_N, _H), jnp.float32)],
        )(accf, hwf, bf[i].reshape(1, _H))[0]
        f = _tc_call(
            _gnorm_body, 6, [jax.ShapeDtypeStruct((_N, _H), jnp.float32)],
        )(convf, pm, mt, fn_w[i].reshape(1, _H), fn_b[i].reshape(1, _H),
          fn_ms[i].reshape(1, _H))[0]

        scale = 2.0 if i == _L - 2 else 1.0
        h, pool = _tc_call(
            _gate_body, 8,
            [jax.ShapeDtypeStruct((_N, _H), jnp.float32),
             jax.ShapeDtypeStruct((_NG, _H), jnp.float32)],
        )(h1, f, h, pm, wg1, wg2, bg.reshape(1, _H), pool / scale)
        pool = pool * scale

    return pool


# speedup vs baseline: 3.3221x; 1.0008x over previous
import functools

import jax
import jax.numpy as jnp
from jax import lax
from jax.experimental import pallas as pl
from jax.experimental.pallas import tpu as pltpu
from jax.experimental.pallas import tpu_sc as plsc

_N = 10000
_H = 128
_NG = 64
_K = 3
_E = 320000
_L = 3
_EPS = 1e-5
_NP = 10240
_R = 400
_C = 512
_NRB = _N // _R
_NCT = _NP // _C
_NW = 32
_NSUB = 16
_CH = 128
_NEG = float('-inf')

_HI = jax.lax.Precision.HIGHEST


def _edges_per_worker(e):
    per = -(-e // _NW)
    nch = -(-per // _CH)
    return nch, _NW * nch * _CH


_NCH_E, _EPAD = _edges_per_worker(_E)
_NCH_F, _FPAD = _edges_per_worker(_N * _K)


def _prep_body(x_ref, we_ref, be_ref, brow_ref, bcol_ref,
               h_ref, hn_ref, pm_ref, mt_ref):
    x = x_ref[...]
    h = jnp.dot(x, we_ref[...], preferred_element_type=jnp.float32,
                precision=jax.lax.Precision.DEFAULT) + be_ref[...]
    h_ref[...] = h
    nrm = jnp.sqrt(jnp.sum(h * h, axis=1, keepdims=True))
    hn_ref[...] = h / jnp.maximum(nrm, 1e-12)
    gids = lax.broadcasted_iota(jnp.int32, (_NG, _N), 0)
    m = (gids == bcol_ref[...]).astype(jnp.float32)
    cnt = jnp.sum(m, axis=1, keepdims=True)
    pm_ref[...] = m / jnp.maximum(cnt, 1.0)
    gidsr = lax.broadcasted_iota(jnp.int32, (_N, _NG), 1)
    mt_ref[...] = (gidsr == brow_ref[...]).astype(jnp.float32)


def _knn_body(s_ref, hn_ref, hnt_ref, brow_ref, bcol_ref, out_ref):
    rb = pl.program_id(0)
    lo = s_ref[0, rb]
    hi = s_ref[1, rb]
    c0 = lo // _C
    c1 = (hi + _C - 1) // _C
    rows = hn_ref[...]
    rbatch = brow_ref[...]
    bv = jnp.full((_R, _K), _NEG, jnp.float32)
    bi = lax.broadcasted_iota(jnp.int32, (_R, _K), 1)

    def tilestep(c, carry):
        bv, bi = carry
        base = c * _C
        cols = hnt_ref[c]
        sim = jnp.dot(rows, cols, preferred_element_type=jnp.float32,
                      precision=jax.lax.Precision.DEFAULT)
        cbatch = bcol_ref[c]
        sim = jnp.where(rbatch == cbatch, sim, _NEG)
        iot = lax.broadcasted_iota(jnp.int32, (_R, _C), 1) + base
        cv, ci = [], []
        for _ in range(_K):
            v = jnp.max(sim, axis=1, keepdims=True)
            sel = jnp.min(jnp.where(sim == v, iot, _NP), axis=1,
                          keepdims=True)
            cv.append(v)
            ci.append(sel)
            sim = jnp.where(iot == sel, _NEG, sim)
        allv = jnp.concatenate([bv] + cv, axis=1)
        alli = jnp.concatenate([bi] + ci, axis=1)
        nv, ni = [], []
        for _ in range(_K):
            v = jnp.max(allv, axis=1, keepdims=True)
            sel = jnp.min(jnp.where(allv == v, alli, _NP), axis=1,
                          keepdims=True)
            nv.append(v)
            ni.append(sel)
            hitm = alli == sel
            allv = jnp.where(hitm, _NEG, allv)
            alli = jnp.where(hitm, _NP, alli)
        return jnp.concatenate(nv, axis=1), jnp.concatenate(ni, axis=1)

    bv, bi = lax.fori_loop(c0, c1, tilestep, (bv, bi))
    out_ref[...] = jnp.concatenate(
        [bi, jnp.zeros((_R, _H - _K), jnp.int32)], axis=1)


def _dinv_from_deg(degp_ref):
    d = 1.0 + degp_ref[0][:_N] + degp_ref[1][:_N]
    return lax.rsqrt(d)


def _prescale_body(h_ref, w_ref, degp_ref, u_ref):
    hw = jnp.dot(h_ref[...], w_ref[...], preferred_element_type=jnp.float32,
                 precision=_HI)
    u = _dinv_from_deg(degp_ref) * hw
    u_ref[...] = jnp.concatenate(
        [u, jnp.zeros((_NP - _N, _H), jnp.float32)], axis=0)


def _graph_norm(h, pm_ref, mt_ref, w, b, ms):
    mean = jnp.dot(pm_ref[...], h, preferred_element_type=jnp.float32,
                   precision=_HI)
    out = h - jnp.dot(mt_ref[...], mean, preferred_element_type=jnp.float32,
                      precision=_HI) * ms
    var = jnp.dot(pm_ref[...], out * out, preferred_element_type=jnp.float32,
                  precision=_HI)
    inv = lax.rsqrt(var + _EPS)
    return w * out * jnp.dot(mt_ref[...], inv,
                             preferred_element_type=jnp.float32,
                             precision=_HI) + b


def _leaky(x):
    return jnp.where(x >= 0, x, 0.01 * x)


def _convsum_body(acc_ref, u_ref, degp_ref, bc_ref, conv_ref):
    s = (acc_ref[0][: _N] + acc_ref[1][: _N] + u_ref[: _N])
    conv_ref[...] = _dinv_from_deg(degp_ref) * s + bc_ref[...]


def _convf_body(accf_ref, hwf_ref, bf_ref, convf_ref):
    convf_ref[...] = 0.25 * (accf_ref[0][: _N] + accf_ref[1][: _N]
                             + hwf_ref[: _N]) + bf_ref[...]


def _gnorm_body(conv_ref, pm_ref, mt_ref, w_ref, b_ref, ms_ref, out_ref):
    out_ref[...] = _leaky(_graph_norm(conv_ref[...], pm_ref, mt_ref,
                                      w_ref[...], b_ref[...], ms_ref[...]))


def _gnorm_mat_body(conv_ref, pm_ref, mt_ref, w_ref, b_ref, ms_ref, wf_ref,
                    h1_ref, hwf_ref):
    h1 = _leaky(_graph_norm(conv_ref[...], pm_ref, mt_ref,
                            w_ref[...], b_ref[...], ms_ref[...]))
    h1_ref[...] = h1
    hwf = jnp.dot(h1, wf_ref[...], preferred_element_type=jnp.float32,
                  precision=_HI)
    hwf_ref[...] = jnp.concatenate(
        [hwf, jnp.zeros((_NP - _N, _H), jnp.float32)], axis=0)


def _gate_body(h1_ref, f_ref, prev_ref, pm_ref, wg1_ref, wg2_ref, bg_ref,
               pool_in_ref, h_ref, pool_ref):
    h1 = h1_ref[...]
    f = f_ref[...]
    z = (jnp.dot(h1, wg1_ref[...], preferred_element_type=jnp.float32,
                 precision=_HI)
         + jnp.dot(f, wg2_ref[...], preferred_element_type=jnp.float32,
                   precision=_HI) + bg_ref[...])
    gate = 1.0 / (1.0 + jnp.exp(-z))
    h = gate * h1 + (1.0 - gate) * f + prev_ref[...]
    h_ref[...] = h
    pool = jnp.dot(pm_ref[...], h, preferred_element_type=jnp.float32,
                   precision=_HI)
    pool_ref[...] = pool_in_ref[...] + pool


def _sc_mesh():
    return plsc.VectorSubcoreMesh(core_axis_name="c", subcore_axis_name="s",
                                  num_cores=2, num_subcores=_NSUB)


_ROWS_PER_SUB = _NP // _NSUB


def _sc_deg_kernel(dst_hbm, ones_hbm, zeros_hbm, out_hbm, idx_v, ones_v,
                   acc_sh):
    cid = lax.axis_index("c")
    sid = lax.axis_index("s")
    w = cid * _NSUB + sid
    pltpu.sync_copy(dst_hbm.at[w], idx_v)
    pltpu.sync_copy(ones_hbm, ones_v)
    pltpu.sync_copy(zeros_hbm.at[pl.ds(sid * _ROWS_PER_SUB, _ROWS_PER_SUB)],
                    acc_sh.at[pl.ds(sid * _ROWS_PER_SUB, _ROWS_PER_SUB)])
    plsc.subcore_barrier()

    def chunk(ch, carry):
        pltpu.sync_copy(ones_v, acc_sh.at[idx_v.at[ch]], add=True)
        return carry

    lax.fori_loop(0, _NCH_E, chunk, 0)
    plsc.subcore_barrier()
    pltpu.sync_copy(acc_sh.at[pl.ds(sid * _ROWS_PER_SUB, _ROWS_PER_SUB)],
                    out_hbm.at[cid, pl.ds(sid * _ROWS_PER_SUB,
                                          _ROWS_PER_SUB)])


def _make_sc_deg():
    return pl.kernel(
        _sc_deg_kernel,
        out_type=jax.ShapeDtypeStruct((2, _NP, _H), jnp.float32),
        mesh=_sc_mesh(),
        scratch_types=[
            pltpu.VMEM((_NCH_E + 1, _CH), jnp.int32),
            pltpu.VMEM((_CH, _H), jnp.float32),
            pltpu.VMEM_SHARED((_NP, _H), jnp.float32),
        ],
    )


def _sc_scatter_kernel(nch, table_hbm, src_hbm, dst_hbm, zeros_hbm, out_hbm,
                       idxs_v, idxd_v, gbuf0, sem0, acc_sh):
    cid = lax.axis_index("c")
    sid = lax.axis_index("s")
    w = cid * _NSUB + sid
    pltpu.sync_copy(src_hbm.at[w], idxs_v)
    pltpu.sync_copy(dst_hbm.at[w], idxd_v)
    pltpu.sync_copy(zeros_hbm.at[pl.ds(sid * _ROWS_PER_SUB, _ROWS_PER_SUB)],
                    acc_sh.at[pl.ds(sid * _ROWS_PER_SUB, _ROWS_PER_SUB)])
    plsc.subcore_barrier()

    def chunk(ch, carry):
        pltpu.async_copy(table_hbm.at[idxs_v.at[ch]], gbuf0, sem0).wait()
        pltpu.sync_copy(gbuf0, acc_sh.at[idxd_v.at[ch]], add=True)
        return carry

    lax.fori_loop(0, nch, chunk, 0)
    plsc.subcore_barrier()
    pltpu.sync_copy(acc_sh.at[pl.ds(sid * _ROWS_PER_SUB, _ROWS_PER_SUB)],
                    out_hbm.at[cid, pl.ds(sid * _ROWS_PER_SUB,
                                          _ROWS_PER_SUB)])


def _make_sc_scatter(nch):
    return pl.kernel(
        functools.partial(_sc_scatter_kernel, nch),
        out_type=jax.ShapeDtypeStruct((2, _NP, _H), jnp.float32),
        mesh=_sc_mesh(),
        scratch_types=[
            pltpu.VMEM((nch + 1, _CH), jnp.int32),
            pltpu.VMEM((nch + 1, _CH), jnp.int32),
            pltpu.VMEM((_CH, _H), jnp.float32),
            pltpu.SemaphoreType.DMA,
            pltpu.VMEM_SHARED((_NP, _H), jnp.float32),
        ],
    )


def _vm(n):
    return [pl.BlockSpec(memory_space=pltpu.VMEM)] * n


def _tc_call(body, n_in, out_shapes):
    return pl.pallas_call(
        body,
        in_specs=_vm(n_in),
        out_specs=[pl.BlockSpec(memory_space=pltpu.VMEM)] * len(out_shapes),
        out_shape=out_shapes,
    )


def _knn_call(sprefetch, hn, hnt3, brow, bcol3):
    grid_spec = pltpu.PrefetchScalarGridSpec(
        num_scalar_prefetch=1,
        grid=(_NRB,),
        in_specs=[
            pl.BlockSpec((_R, _H), lambda i, s: (i, 0)),
            pl.BlockSpec((_NCT, _H, _C), lambda i, s: (0, 0, 0)),
            pl.BlockSpec((_R, 1), lambda i, s: (i, 0)),
            pl.BlockSpec((_NCT, 1, _C), lambda i, s: (0, 0, 0)),
        ],
        out_specs=pl.BlockSpec((_R, _H), lambda i, s: (i, 0)),
    )
    return pl.pallas_call(
        _knn_body,
        grid_spec=grid_spec,
        out_shape=jax.ShapeDtypeStruct((_N, _H), jnp.int32),
    )(sprefetch, hn, hnt3, brow, bcol3)


def _pad_edges(e_arr, nch):
    npad = _NW * nch * _CH - e_arr.shape[0]
    filler = _N + (jnp.arange(npad, dtype=jnp.int32) % (_NP - _N))
    mat = jnp.concatenate([e_arr, filler]).reshape(_NW, nch, _CH)
    junk = jnp.full((_NW, 1, _CH), _N, jnp.int32)
    return jnp.concatenate([mat, junk], axis=1)


def kernel(x, edge_index, batch, W_emb, b_emb, Wc, bc, gn_w, gn_b, gn_ms,
           Wf, bf, fn_w, fn_b, fn_ms, Wg, bg):
    src = edge_index[0]
    dst = edge_index[1]
    brow = batch.reshape(_N, 1)
    bcol = batch.reshape(1, _N)

    dst3 = _pad_edges(dst, _NCH_E)
    src3 = _pad_edges(src, _NCH_E)

    zeros128 = jnp.zeros((_NP, _H), jnp.float32)
    ones128 = jnp.ones((_CH, _H), jnp.float32)

    degp = _make_sc_deg()(dst3, ones128, zeros128)
    degp_sl = degp[:, :, :1]

    h0, hn, pm, mt = _tc_call(
        _prep_body, 5,
        [jax.ShapeDtypeStruct((_N, _H), jnp.float32),
         jax.ShapeDtypeStruct((_N, _H), jnp.float32),
         jax.ShapeDtypeStruct((_NG, _N), jnp.float32),
         jax.ShapeDtypeStruct((_N, _NG), jnp.float32)],
    )(x, W_emb, b_emb.reshape(1, _H), brow, bcol)

    bs = batch[0::_R]
    be = batch[_R - 1::_R]
    lo = jnp.searchsorted(batch, bs, side="left").astype(jnp.int32)
    hi = jnp.searchsorted(batch, be, side="right").astype(jnp.int32)
    sprefetch = jnp.stack([lo, hi])

    hnp = jnp.concatenate([hn, jnp.zeros((_NP - _N, _H), jnp.float32)])
    hnt3 = hnp.reshape(_NCT, _C, _H).transpose(0, 2, 1)
    bcolp = jnp.concatenate(
        [batch, jnp.full((_NP - _N,), -1, jnp.int32)]).reshape(_NCT, 1, _C)

    idx_wide = _knn_call(sprefetch, hn, hnt3, brow, bcolp)
    fsrc = idx_wide[:, :_K].reshape(-1)
    fsrc3 = _pad_edges(fsrc, _NCH_F)
    fdst = jnp.repeat(jnp.arange(_N, dtype=jnp.int32), _K)
    fdst3 = _pad_edges(fdst, _NCH_F)

    sc_edge = _make_sc_scatter(_NCH_E)
    sc_feat = _make_sc_scatter(_NCH_F)

    wg1 = Wg[:_H]
    wg2 = Wg[_H:]

    h = h0
    pool = jnp.zeros((_NG, _H), jnp.float32)
    for i in range(_L - 1):
        u = _tc_call(
            _prescale_body, 3,
            [jax.ShapeDtypeStruct((_NP, _H), jnp.float32)],
        )(h, Wc[i], degp_sl)[0]

        acc = sc_edge(u, src3, dst3, zeros128)

        conv = _tc_call(
            _convsum_body, 4, [jax.ShapeDtypeStruct((_N, _H), jnp.float32)],
        )(acc, u, degp_sl, bc[i].reshape(1, _H))[0]
        h1, hwf = _tc_call(
            _gnorm_mat_body, 7,
            [jax.ShapeDtypeStruct((_N, _H), jnp.float32),
             jax.ShapeDtypeStruct((_NP, _H), jnp.float32)],
        )(conv, pm, mt, gn_w[i].reshape(1, _H), gn_b[i].reshape(1, _H),
          gn_ms[i].reshape(1, _H), Wf[i])

        accf = sc_feat(hwf, fsrc3, fdst3, zeros128)

        convf = _tc_call(
            _convf_body, 3, [jax.ShapeDtypeStruct((_N, _H), jnp.float32)],
        )(accf, hwf, bf[i].reshape(1, _H))[0]
        f = _tc_call(
            _gnorm_body, 6, [jax.ShapeDtypeStruct((_N, _H), jnp.float32)],
        )(convf, pm, mt, fn_w[i].reshape(1, _H), fn_b[i].reshape(1, _H),
          fn_ms[i].reshape(1, _H))[0]

        scale = 2.0 if i == _L - 2 else 1.0
        h, pool = _tc_call(
            _gate_body, 8,
            [jax.ShapeDtypeStruct((_N, _H), jnp.float32),
             jax.ShapeDtypeStruct((_NG, _H), jnp.float32)],
        )(h1, f, h, pm, wg1, wg2, bg.reshape(1, _H), pool / scale)
        pool = pool * scale

    return pool
